# Initial kernel scaffold; baseline (speedup 1.0000x reference)
import jax
import jax.numpy as jnp
from jax.experimental import pallas as pl


def kernel(xyz, points, Ws, bs):
    raise NotImplementedError("write your pallas kernel here")


# speedup vs baseline: 1.0012x; 1.0012x over previous
import jax
import jax.numpy as jnp
import numpy as np
from jax.experimental import pallas as pl

_NPOINT = 512
_RADIUS_LIST = [0.1, 0.2, 0.4]
_NSAMPLE_LIST = [16, 32, 64]
_IN_CHANNEL = 64


def _square_distance(src, dst):
    dist = -2.0 * jnp.matmul(src, jnp.swapaxes(dst, 1, 2))
    dist = dist + jnp.sum(src ** 2, -1)[:, :, None]
    dist = dist + jnp.sum(dst ** 2, -1)[:, None, :]
    return dist


def _index_points(points, idx):
    return jax.vmap(lambda p, i: p[i])(points, idx)


def _farthest_point_sample(xyz, npoint):
    xyz = jax.lax.stop_gradient(xyz)
    B, N, _ = xyz.shape

    def body(i, carry):
        centroids, distance, farthest = carry
        centroids = centroids.at[:, i].set(farthest)
        centroid = xyz[jnp.arange(B), farthest, :][:, None, :]
        dist = jnp.sum((xyz - centroid) ** 2, -1)
        distance = jnp.minimum(distance, dist)
        farthest = jnp.argmax(distance, -1).astype(jnp.int32)
        return centroids, distance, farthest

    init = (jnp.zeros((B, npoint), jnp.int32),
            jnp.full((B, N), 1e10, dtype=jnp.float32),
            jnp.zeros((B,), jnp.int32))
    centroids, _, _ = jax.lax.fori_loop(0, npoint, body, init)
    return centroids


def _query_ball_point(radius, nsample, xyz, new_xyz):
    B, N, _ = xyz.shape
    S = new_xyz.shape[1]
    group_idx = jnp.broadcast_to(jnp.arange(N, dtype=jnp.int32)[None, None, :], (B, S, N))
    sqrdists = _square_distance(new_xyz, xyz)
    group_idx = jnp.where(sqrdists > radius ** 2, N, group_idx)
    group_idx = jnp.sort(group_idx, axis=-1)[:, :, :nsample]
    group_first = jnp.broadcast_to(group_idx[:, :, 0:1], group_idx.shape)
    group_idx = jnp.where(group_idx == N, group_first, group_idx)
    return group_idx


def _identity_kernel(x_ref, o_ref):
    o_ref[...] = x_ref[...]


def _pallas_identity(x):
    return pl.pallas_call(
        _identity_kernel,
        out_shape=jax.ShapeDtypeStruct(x.shape, x.dtype),
    )(x)


def kernel(xyz, points, Ws, bs):
    xyz_t = jnp.transpose(xyz, (0, 2, 1))
    pts_t = jnp.transpose(points, (0, 2, 1))
    B, N, C = xyz_t.shape
    S = _NPOINT
    fps_idx = _farthest_point_sample(xyz_t, S)
    new_xyz = _index_points(xyz_t, fps_idx)
    new_points_list = []
    for i, radius in enumerate(_RADIUS_LIST):
        K = _NSAMPLE_LIST[i]
        group_idx = _query_ball_point(radius, K, xyz_t, new_xyz)
        grouped_xyz = _index_points(xyz_t, group_idx)
        grouped_xyz = grouped_xyz - new_xyz[:, :, None, :]
        grouped_points = _index_points(pts_t, group_idx)
        grouped_points = jnp.concatenate([grouped_points, grouped_xyz], axis=-1)
        h = grouped_points
        for W, b in zip(Ws[i], bs[i]):
            h = jax.nn.relu(jnp.einsum('bskc,oc->bsko', h, W) + b)
        new_points = jnp.max(h, axis=2)
        new_points_list.append(jnp.transpose(new_points, (0, 2, 1)))
    new_xyz_out = jnp.transpose(new_xyz, (0, 2, 1))
    new_points_concat = jnp.concatenate(new_points_list, axis=1)
    new_xyz_out = _pallas_identity(new_xyz_out)
    return new_xyz_out, new_points_concat


# speedup vs baseline: 1.4338x; 1.4322x over previous
import jax
import jax.numpy as jnp
import numpy as np
from jax.experimental import pallas as pl

_NPOINT = 512
_RADIUS_LIST = [0.1, 0.2, 0.4]
_NSAMPLE_LIST = [16, 32, 64]
_IN_CHANNEL = 64


def _square_distance(src, dst):
    dist = -2.0 * jnp.matmul(src, jnp.swapaxes(dst, 1, 2))
    dist = dist + jnp.sum(src ** 2, -1)[:, :, None]
    dist = dist + jnp.sum(dst ** 2, -1)[:, None, :]
    return dist


def _index_points(points, idx):
    return jax.vmap(lambda p, i: p[i])(points, idx)


def _query_ball_point(radius, nsample, xyz, new_xyz):
    B, N, _ = xyz.shape
    S = new_xyz.shape[1]
    group_idx = jnp.broadcast_to(jnp.arange(N, dtype=jnp.int32)[None, None, :], (B, S, N))
    sqrdists = _square_distance(new_xyz, xyz)
    group_idx = jnp.where(sqrdists > radius ** 2, N, group_idx)
    group_idx = jnp.sort(group_idx, axis=-1)[:, :, :nsample]
    group_first = jnp.broadcast_to(group_idx[:, :, 0:1], group_idx.shape)
    group_idx = jnp.where(group_idx == N, group_first, group_idx)
    return group_idx


def _fps_kernel(x_ref, y_ref, z_ref, ox_ref, oy_ref, oz_ref):
    B, N = x_ref.shape
    S = ox_ref.shape[1]
    iota_f = jax.lax.broadcasted_iota(jnp.int32, (B, N), 1).astype(jnp.float32)
    iota_s = jax.lax.broadcasted_iota(jnp.int32, (B, S), 1)

    def body(i, carry):
        distance, farthest, cxs, cys, czs = carry
        x = x_ref[...]
        y = y_ref[...]
        z = z_ref[...]
        onehot = (iota_f == farthest).astype(jnp.float32)
        cx = jnp.sum(x * onehot, axis=1, keepdims=True)
        cy = jnp.sum(y * onehot, axis=1, keepdims=True)
        cz = jnp.sum(z * onehot, axis=1, keepdims=True)
        slot = (iota_s == i).astype(jnp.float32)
        cxs = cxs + slot * cx
        cys = cys + slot * cy
        czs = czs + slot * cz
        dx = x - cx
        dy = y - cy
        dz = z - cz
        dist = (dx * dx + dy * dy) + dz * dz
        distance = jnp.minimum(distance, dist)
        maxv = jnp.max(distance, axis=1, keepdims=True)
        farthest = jnp.min(
            jnp.where(distance == maxv, iota_f, float(N)),
            axis=1, keepdims=True)
        return distance, farthest, cxs, cys, czs

    zs = jnp.zeros((B, S), jnp.float32)
    init = (jnp.full((B, N), 1e10, jnp.float32),
            jnp.zeros((B, 1), jnp.float32),
            zs, zs, zs)
    _, _, cxs, cys, czs = jax.lax.fori_loop(0, S, body, init)
    ox_ref[...] = cxs
    oy_ref[...] = cys
    oz_ref[...] = czs


def _fps_pallas(xyz, npoint):
    B, _, N = xyz.shape
    ox, oy, oz = pl.pallas_call(
        _fps_kernel,
        out_shape=(
            jax.ShapeDtypeStruct((B, npoint), jnp.float32),
            jax.ShapeDtypeStruct((B, npoint), jnp.float32),
            jax.ShapeDtypeStruct((B, npoint), jnp.float32),
        ),
    )(xyz[:, 0], xyz[:, 1], xyz[:, 2])
    return jnp.stack([ox, oy, oz], axis=1)


def kernel(xyz, points, Ws, bs):
    xyz_t = jnp.transpose(xyz, (0, 2, 1))
    pts_t = jnp.transpose(points, (0, 2, 1))
    B, N, C = xyz_t.shape
    S = _NPOINT
    new_xyz_cs = _fps_pallas(xyz, S)
    new_xyz = jnp.transpose(new_xyz_cs, (0, 2, 1))
    new_points_list = []
    for i, radius in enumerate(_RADIUS_LIST):
        K = _NSAMPLE_LIST[i]
        group_idx = _query_ball_point(radius, K, xyz_t, new_xyz)
        grouped_xyz = _index_points(xyz_t, group_idx)
        grouped_xyz = grouped_xyz - new_xyz[:, :, None, :]
        grouped_points = _index_points(pts_t, group_idx)
        grouped_points = jnp.concatenate([grouped_points, grouped_xyz], axis=-1)
        h = grouped_points
        for W, b in zip(Ws[i], bs[i]):
            h = jax.nn.relu(jnp.einsum('bskc,oc->bsko', h, W) + b)
        new_points = jnp.max(h, axis=2)
        new_points_list.append(jnp.transpose(new_points, (0, 2, 1)))
    new_points_concat = jnp.concatenate(new_points_list, axis=1)
    return new_xyz_cs, new_points_concat


# speedup vs baseline: 2.3191x; 1.6174x over previous
import dataclasses
import functools

import jax
import jax.numpy as jnp
import numpy as np
from jax import lax
from jax.experimental import pallas as pl
from jax.experimental.pallas import tpu as pltpu
from jax.experimental.pallas import tpu_sc as plsc

_NPOINT = 512
_RADIUS_LIST = [0.1, 0.2, 0.4]
_NSAMPLE_LIST = [16, 32, 64]
_IN_CHANNEL = 64


def _square_distance(src, dst):
    dist = -2.0 * jnp.matmul(src, jnp.swapaxes(dst, 1, 2))
    dist = dist + jnp.sum(src ** 2, -1)[:, :, None]
    dist = dist + jnp.sum(dst ** 2, -1)[:, None, :]
    return dist


def _index_points(points, idx):
    return jax.vmap(lambda p, i: p[i])(points, idx)


def _pack_matrix(n_total):
    n = np.arange(n_total)
    w = n // 32
    j = (n % 32) // 16
    p = n % 16
    P = np.zeros((n_total, 256), np.float32)
    P[n, j * 128 + w % 128] = (2.0 ** p)
    return jnp.asarray(P, jnp.bfloat16)


_NCHUNK = 512


def _ballmask_kernel(x_ref, y_ref, z_ref, cx_ref, cy_ref, cz_ref, p_ref,
                     w0_ref, w1_ref, w2_ref, acc0, acc1, acc2):
    nc = pl.program_id(2)
    n_steps = pl.num_programs(2)

    ii = jax.lax.broadcasted_iota(jnp.int32, (128, 128), 0)
    jj = jax.lax.broadcasted_iota(jnp.int32, (128, 128), 1)
    diag = (ii == jj).astype(jnp.float32)

    def col(row_ref):
        a = jnp.broadcast_to(row_ref[0], (128, 128))
        return jnp.sum(a * diag, axis=1, keepdims=True)

    cx = col(cx_ref)
    cy = col(cy_ref)
    cz = col(cz_ref)
    x = x_ref[0]
    y = y_ref[0]
    z = z_ref[0]
    def tr(v):
        return v.astype(jnp.bfloat16).astype(jnp.float32)

    cross = (tr(x) * tr(cx) + tr(y) * tr(cy)) + tr(z) * tr(cz)
    c2 = (cx * cx + cy * cy) + cz * cz
    x2 = (x * x + y * y) + z * z
    d = ((-2.0) * cross + c2) + x2
    p = p_ref[...]

    @pl.when(nc == 0)
    def _init():
        acc0[...] = jnp.zeros_like(acc0)
        acc1[...] = jnp.zeros_like(acc1)
        acc2[...] = jnp.zeros_like(acc2)

    for r2, acc in ((_RADIUS_LIST[0] ** 2, acc0),
                    (_RADIUS_LIST[1] ** 2, acc1),
                    (_RADIUS_LIST[2] ** 2, acc2)):
        m = (d <= r2).astype(jnp.bfloat16)
        acc[...] += jnp.dot(m, p, preferred_element_type=jnp.float32)

    @pl.when(nc == n_steps - 1)
    def _emit():
        for acc, w_ref in ((acc0, w0_ref), (acc1, w1_ref), (acc2, w2_ref)):
            v = acc[...].astype(jnp.int32)
            lo = v[:, :128]
            hi = v[:, 128:]
            w_ref[0, :, :] = lo + (hi << 16)


def _ballmask_pallas(x, y, z, cxs, cys, czs):
    B, N = x.shape
    S = cxs.shape[1]
    P = _pack_matrix(N)
    grid = (B, S // 128, N // _NCHUNK)
    bs_n = pl.BlockSpec((1, 1, _NCHUNK), lambda b, st, nc: (b, 0, nc))
    bs_s = pl.BlockSpec((1, 1, 128), lambda b, st, nc: (b, 0, st))
    bs_p = pl.BlockSpec((_NCHUNK, 256), lambda b, st, nc: (nc, 0))
    bs_w = pl.BlockSpec((1, 128, 128), lambda b, st, nc: (b, st, 0))
    out = jax.ShapeDtypeStruct((B, S, 128), jnp.int32)
    x3 = x[:, None, :]
    y3 = y[:, None, :]
    z3 = z[:, None, :]
    return pl.pallas_call(
        _ballmask_kernel,
        grid=grid,
        in_specs=[bs_n, bs_n, bs_n, bs_s, bs_s, bs_s, bs_p],
        out_specs=[bs_w, bs_w, bs_w],
        out_shape=(out, out, out),
        scratch_shapes=[pltpu.VMEM((128, 256), jnp.float32)] * 3,
    )(x3, y3, z3, cxs[:, None, :], cys[:, None, :], czs[:, None, :], P)


def _swar_popcount(w):
    c1 = jnp.int32(0x55555555)
    c2 = jnp.int32(0x33333333)
    c4 = jnp.int32(0x0F0F0F0F)
    w = w - (lax.shift_right_logical(w, 1) & c1)
    w = (w & c2) + (lax.shift_right_logical(w, 2) & c2)
    w = (w + lax.shift_right_logical(w, 4)) & c4
    return lax.shift_right_logical(w * jnp.int32(0x01010101), 24)


_ROWS_BLK = 16


def _sc_select_one(wbuf, obuf, k, r):
    iota16 = lax.iota(jnp.int32, 16)
    rvec = jnp.full((16,), r, jnp.int32)

    def chunk_fn(c, base):
        def go(base):
            w0 = wbuf[r, pl.ds(c * 16, 16)]
            pc = _swar_popcount(w0)
            incl = plsc.cumsum(pc)
            rank0 = (base + incl) - pc
            total = base + jnp.max(incl)

            def cond(st):
                w, _ = st
                return jnp.any(w != 0)

            def body(st):
                w, rank = st
                emit = w != 0
                lsb = w & (-w)
                f = plsc.bitcast(lsb.astype(jnp.float32), jnp.int32)
                posl = lax.shift_right_logical(
                    f & jnp.int32(0x7FFFFFFF), 23) - 127
                n = ((c * 16 + iota16) << 5) + posl
                slots = jnp.minimum(rank, k - 1)
                plsc.store_scatter(obuf, [rvec, slots], n, mask=emit)
                rank = rank + emit.astype(jnp.int32)
                w = w & (w - 1)
                w = jnp.where(rank < k, w, 0)
                return w, rank

            w0c = jnp.where(rank0 < k, w0, jnp.zeros((16,), jnp.int32))
            lax.while_loop(cond, body, (w0c, rank0))
            return total

        return lax.cond(base < k, go, lambda b: b, base)

    count = lax.fori_loop(0, 8, chunk_fn, jnp.int32(0))
    count = jnp.minimum(count, k)
    first = plsc.load_gather(obuf, [rvec, jnp.zeros((16,), jnp.int32)])
    for j in range(k // 16):
        slots = iota16 + 16 * j
        cur = obuf[r, pl.ds(16 * j, 16)]
        obuf[r, pl.ds(16 * j, 16)] = jnp.where(slots < count, cur, first)


def _ballquery_sc(w0, w1, w2):
    R = w0.shape[0]
    mesh = plsc.VectorSubcoreMesh(core_axis_name="c", subcore_axis_name="s")
    nwork = 32
    rows_per = R // nwork
    cp = pltpu.CompilerParams()
    if "needs_layout_passes" in pltpu.CompilerParams.__dataclass_fields__:
        cp = dataclasses.replace(cp, needs_layout_passes=False)

    @functools.partial(
        pl.kernel,
        compiler_params=cp,
        out_type=(
            jax.ShapeDtypeStruct((R, 16), jnp.int32),
            jax.ShapeDtypeStruct((R, 32), jnp.int32),
            jax.ShapeDtypeStruct((R, 64), jnp.int32),
        ),
        mesh=mesh,
        scratch_types=[
            pltpu.VMEM((_ROWS_BLK, 128), jnp.int32),
            pltpu.VMEM((_ROWS_BLK, 16), jnp.int32),
            pltpu.VMEM((_ROWS_BLK, 32), jnp.int32),
            pltpu.VMEM((_ROWS_BLK, 64), jnp.int32),
        ],
    )
    def sckern(w0_hbm, w1_hbm, w2_hbm, o0_hbm, o1_hbm, o2_hbm,
               wbuf, ob16, ob32, ob64):
        wid = lax.axis_index("s") * 2 + lax.axis_index("c")
        row_base = wid * rows_per
        for w_hbm, o_hbm, obuf, k in ((w0_hbm, o0_hbm, ob16, 16),
                                      (w1_hbm, o1_hbm, ob32, 32),
                                      (w2_hbm, o2_hbm, ob64, 64)):
            for blk in range(rows_per // _ROWS_BLK):
                row0 = row_base + blk * _ROWS_BLK
                pltpu.sync_copy(w_hbm.at[pl.ds(row0, _ROWS_BLK), :], wbuf)

                def row_fn(r, _, obuf=obuf, k=k):
                    _sc_select_one(wbuf, obuf, k, r)
                    return 0

                lax.fori_loop(0, _ROWS_BLK, row_fn, 0)
                pltpu.sync_copy(obuf, o_hbm.at[pl.ds(row0, _ROWS_BLK), :])

    return sckern(w0, w1, w2)


def _fps_kernel(x_ref, y_ref, z_ref, ox_ref, oy_ref, oz_ref):
    B, N = x_ref.shape
    S = ox_ref.shape[1]
    iota_f = jax.lax.broadcasted_iota(jnp.int32, (B, N), 1).astype(jnp.float32)
    iota_s = jax.lax.broadcasted_iota(jnp.int32, (B, S), 1)

    def body(i, carry):
        distance, farthest, cxs, cys, czs = carry
        x = x_ref[...]
        y = y_ref[...]
        z = z_ref[...]
        onehot = (iota_f == farthest).astype(jnp.float32)
        cx = jnp.sum(x * onehot, axis=1, keepdims=True)
        cy = jnp.sum(y * onehot, axis=1, keepdims=True)
        cz = jnp.sum(z * onehot, axis=1, keepdims=True)
        slot = (iota_s == i).astype(jnp.float32)
        cxs = cxs + slot * cx
        cys = cys + slot * cy
        czs = czs + slot * cz
        dx = x - cx
        dy = y - cy
        dz = z - cz
        dist = (dx * dx + dy * dy) + dz * dz
        distance = jnp.minimum(distance, dist)
        maxv = jnp.max(distance, axis=1, keepdims=True)
        farthest = jnp.min(
            jnp.where(distance == maxv, iota_f, float(N)),
            axis=1, keepdims=True)
        return distance, farthest, cxs, cys, czs

    zs = jnp.zeros((B, S), jnp.float32)
    init = (jnp.full((B, N), 1e10, jnp.float32),
            jnp.zeros((B, 1), jnp.float32),
            zs, zs, zs)
    _, _, cxs, cys, czs = jax.lax.fori_loop(0, S, body, init)
    ox_ref[...] = cxs
    oy_ref[...] = cys
    oz_ref[...] = czs


def _fps_pallas(xyz, npoint):
    B, _, N = xyz.shape
    ox, oy, oz = pl.pallas_call(
        _fps_kernel,
        out_shape=(
            jax.ShapeDtypeStruct((B, npoint), jnp.float32),
            jax.ShapeDtypeStruct((B, npoint), jnp.float32),
            jax.ShapeDtypeStruct((B, npoint), jnp.float32),
        ),
    )(xyz[:, 0], xyz[:, 1], xyz[:, 2])
    return ox, oy, oz


def kernel(xyz, points, Ws, bs):
    xyz_t = jnp.transpose(xyz, (0, 2, 1))
    pts_t = jnp.transpose(points, (0, 2, 1))
    B, N, C = xyz_t.shape
    S = _NPOINT
    cxs, cys, czs = _fps_pallas(xyz, S)
    new_xyz_cs = jnp.stack([cxs, cys, czs], axis=1)
    new_xyz = jnp.transpose(new_xyz_cs, (0, 2, 1))
    w0, w1, w2 = _ballmask_pallas(
        xyz[:, 0], xyz[:, 1], xyz[:, 2], cxs, cys, czs)
    g0, g1, g2 = _ballquery_sc(
        w0.reshape(B * S, 128), w1.reshape(B * S, 128),
        w2.reshape(B * S, 128))
    group_idx_all = [g0.reshape(B, S, 16), g1.reshape(B, S, 32),
                     g2.reshape(B, S, 64)]
    new_points_list = []
    for i, radius in enumerate(_RADIUS_LIST):
        K = _NSAMPLE_LIST[i]
        group_idx = group_idx_all[i]
        grouped_xyz = _index_points(xyz_t, group_idx)
        grouped_xyz = grouped_xyz - new_xyz[:, :, None, :]
        grouped_points = _index_points(pts_t, group_idx)
        grouped_points = jnp.concatenate([grouped_points, grouped_xyz], axis=-1)
        h = grouped_points
        for W, b in zip(Ws[i], bs[i]):
            h = jax.nn.relu(jnp.einsum('bskc,oc->bsko', h, W) + b)
        new_points = jnp.max(h, axis=2)
        new_points_list.append(jnp.transpose(new_points, (0, 2, 1)))
    new_points_concat = jnp.concatenate(new_points_list, axis=1)
    return new_xyz_cs, new_points_concat


# speedup vs baseline: 16.6681x; 7.1872x over previous
import dataclasses
import functools

import jax
import jax.numpy as jnp
import numpy as np
from jax import lax
from jax.experimental import pallas as pl
from jax.experimental.pallas import tpu as pltpu
from jax.experimental.pallas import tpu_sc as plsc

_NPOINT = 512
_RADIUS_LIST = [0.1, 0.2, 0.4]
_NSAMPLE_LIST = [16, 32, 64]
_IN_CHANNEL = 64


def _square_distance(src, dst):
    dist = -2.0 * jnp.matmul(src, jnp.swapaxes(dst, 1, 2))
    dist = dist + jnp.sum(src ** 2, -1)[:, :, None]
    dist = dist + jnp.sum(dst ** 2, -1)[:, None, :]
    return dist


def _index_points(points, idx):
    return jax.vmap(lambda p, i: p[i])(points, idx)


def _pack_matrix(n_total):
    n = np.arange(n_total)
    w = n // 32
    j = (n % 32) // 16
    p = n % 16
    P = np.zeros((n_total, 256), np.float32)
    P[n, j * 128 + w % 128] = (2.0 ** p)
    return jnp.asarray(P, jnp.bfloat16)


_NCHUNK = 512


def _ballmask_kernel(x_ref, y_ref, z_ref, cx_ref, cy_ref, cz_ref, p_ref,
                     w0_ref, w1_ref, w2_ref, acc0, acc1, acc2):
    nc = pl.program_id(2)
    n_steps = pl.num_programs(2)

    ii = jax.lax.broadcasted_iota(jnp.int32, (128, 128), 0)
    jj = jax.lax.broadcasted_iota(jnp.int32, (128, 128), 1)
    diag = (ii == jj).astype(jnp.float32)

    def col(row_ref):
        a = jnp.broadcast_to(row_ref[0], (128, 128))
        return jnp.sum(a * diag, axis=1, keepdims=True)

    cx = col(cx_ref)
    cy = col(cy_ref)
    cz = col(cz_ref)
    x = x_ref[0]
    y = y_ref[0]
    z = z_ref[0]
    def tr(v):
        return v.astype(jnp.bfloat16).astype(jnp.float32)

    p0 = tr(x) * tr(cx)
    p1 = tr(y) * tr(cy)
    p2 = tr(z) * tr(cz)
    s = p0 + p1
    bb = s - p0
    e1 = (p0 - (s - bb)) + (p1 - bb)
    t = s + p2
    bb2 = t - s
    e2 = (s - (t - bb2)) + (p2 - bb2)
    cross = t + (e1 + e2)
    c2 = (cx * cx + cy * cy) + cz * cz
    x2 = (x * x + y * y) + z * z
    d = ((-2.0) * cross + c2) + x2
    p = p_ref[...]

    @pl.when(nc == 0)
    def _init():
        acc0[...] = jnp.zeros_like(acc0)
        acc1[...] = jnp.zeros_like(acc1)
        acc2[...] = jnp.zeros_like(acc2)

    for r2, acc in ((_RADIUS_LIST[0] ** 2, acc0),
                    (_RADIUS_LIST[1] ** 2, acc1),
                    (_RADIUS_LIST[2] ** 2, acc2)):
        m = (d <= r2).astype(jnp.bfloat16)
        acc[...] += jnp.dot(m, p, preferred_element_type=jnp.float32)

    @pl.when(nc == n_steps - 1)
    def _emit():
        for acc, w_ref in ((acc0, w0_ref), (acc1, w1_ref), (acc2, w2_ref)):
            v = acc[...].astype(jnp.int32)
            lo = v[:, :128]
            hi = v[:, 128:]
            w_ref[0, :, :] = lo + (hi << 16)


def _ballmask_pallas(x, y, z, cxs, cys, czs):
    B, N = x.shape
    S = cxs.shape[1]
    P = _pack_matrix(N)
    grid = (B, S // 128, N // _NCHUNK)
    bs_n = pl.BlockSpec((1, 1, _NCHUNK), lambda b, st, nc: (b, 0, nc))
    bs_s = pl.BlockSpec((1, 1, 128), lambda b, st, nc: (b, 0, st))
    bs_p = pl.BlockSpec((_NCHUNK, 256), lambda b, st, nc: (nc, 0))
    bs_w = pl.BlockSpec((1, 128, 128), lambda b, st, nc: (b, st, 0))
    out = jax.ShapeDtypeStruct((B, S, 128), jnp.int32)
    x3 = x[:, None, :]
    y3 = y[:, None, :]
    z3 = z[:, None, :]
    return pl.pallas_call(
        _ballmask_kernel,
        grid=grid,
        in_specs=[bs_n, bs_n, bs_n, bs_s, bs_s, bs_s, bs_p],
        out_specs=[bs_w, bs_w, bs_w],
        out_shape=(out, out, out),
        scratch_shapes=[pltpu.VMEM((128, 256), jnp.float32)] * 3,
    )(x3, y3, z3, cxs[:, None, :], cys[:, None, :], czs[:, None, :], P)


def _swar_popcount(w):
    c1 = jnp.int32(0x55555555)
    c2 = jnp.int32(0x33333333)
    c4 = jnp.int32(0x0F0F0F0F)
    w = w - (lax.shift_right_logical(w, 1) & c1)
    w = (w & c2) + (lax.shift_right_logical(w, 2) & c2)
    w = (w + lax.shift_right_logical(w, 4)) & c4
    return lax.shift_right_logical(w * jnp.int32(0x01010101), 24)


_ROWS_BLK = 16


def _sc_select_one(wbuf, obuf, k, r):
    iota16 = lax.iota(jnp.int32, 16)
    rvec = jnp.full((16,), r, jnp.int32)

    def chunk_fn(c, base):
        def go(base):
            w0 = wbuf[r, pl.ds(c * 16, 16)]
            pc = _swar_popcount(w0)
            incl = plsc.cumsum(pc)
            rank0 = (base + incl) - pc
            total = base + jnp.max(incl)

            def cond(st):
                w, _ = st
                return jnp.any(w != 0)

            def body(st):
                w, rank = st
                emit = w != 0
                lsb = w & (-w)
                f = plsc.bitcast(lsb.astype(jnp.float32), jnp.int32)
                posl = lax.shift_right_logical(
                    f & jnp.int32(0x7FFFFFFF), 23) - 127
                n = ((c * 16 + iota16) << 5) + posl
                slots = jnp.minimum(rank, k - 1)
                plsc.store_scatter(obuf, [rvec, slots], n, mask=emit)
                rank = rank + emit.astype(jnp.int32)
                w = w & (w - 1)
                w = jnp.where(rank < k, w, 0)
                return w, rank

            w0c = jnp.where(rank0 < k, w0, jnp.zeros((16,), jnp.int32))
            lax.while_loop(cond, body, (w0c, rank0))
            return total

        return lax.cond(base < k, go, lambda b: b, base)

    count = lax.fori_loop(0, 8, chunk_fn, jnp.int32(0))
    count = jnp.minimum(count, k)
    first = plsc.load_gather(obuf, [rvec, jnp.zeros((16,), jnp.int32)])
    m = (count > 0).astype(jnp.int32)
    fillv = first * m + (1 - m) * jnp.int32(4095)
    for j in range(k // 16):
        slots = iota16 + 16 * j
        cur = obuf[r, pl.ds(16 * j, 16)]
        obuf[r, pl.ds(16 * j, 16)] = jnp.where(slots < count, cur, fillv)


def _ballquery_sc(w0, w1, w2):
    R = w0.shape[0]
    mesh = plsc.VectorSubcoreMesh(core_axis_name="c", subcore_axis_name="s")
    nwork = 32
    rows_per = R // nwork
    cp = pltpu.CompilerParams()
    if "needs_layout_passes" in pltpu.CompilerParams.__dataclass_fields__:
        cp = dataclasses.replace(cp, needs_layout_passes=False)

    @functools.partial(
        pl.kernel,
        compiler_params=cp,
        out_type=(
            jax.ShapeDtypeStruct((R, 16), jnp.int32),
            jax.ShapeDtypeStruct((R, 32), jnp.int32),
            jax.ShapeDtypeStruct((R, 64), jnp.int32),
        ),
        mesh=mesh,
        scratch_types=[
            pltpu.VMEM((_ROWS_BLK, 128), jnp.int32),
            pltpu.VMEM((_ROWS_BLK, 16), jnp.int32),
            pltpu.VMEM((_ROWS_BLK, 32), jnp.int32),
            pltpu.VMEM((_ROWS_BLK, 64), jnp.int32),
        ],
    )
    def sckern(w0_hbm, w1_hbm, w2_hbm, o0_hbm, o1_hbm, o2_hbm,
               wbuf, ob16, ob32, ob64):
        wid = lax.axis_index("s") * 2 + lax.axis_index("c")
        row_base = wid * rows_per
        for w_hbm, o_hbm, obuf, k in ((w0_hbm, o0_hbm, ob16, 16),
                                      (w1_hbm, o1_hbm, ob32, 32),
                                      (w2_hbm, o2_hbm, ob64, 64)):
            for blk in range(rows_per // _ROWS_BLK):
                row0 = row_base + blk * _ROWS_BLK
                pltpu.sync_copy(w_hbm.at[pl.ds(row0, _ROWS_BLK), :], wbuf)

                def row_fn(r, _, obuf=obuf, k=k):
                    _sc_select_one(wbuf, obuf, k, r)
                    return 0

                lax.fori_loop(0, _ROWS_BLK, row_fn, 0)
                pltpu.sync_copy(obuf, o_hbm.at[pl.ds(row0, _ROWS_BLK), :])

    return sckern(w0, w1, w2)


def _fps_kernel(x_ref, y_ref, z_ref, ox_ref, oy_ref, oz_ref):
    B, N = x_ref.shape
    S = ox_ref.shape[1]
    iota_f = jax.lax.broadcasted_iota(jnp.int32, (B, N), 1).astype(jnp.float32)
    iota_s = jax.lax.broadcasted_iota(jnp.int32, (B, S), 1)

    def body(i, carry):
        distance, farthest, cxs, cys, czs = carry
        x = x_ref[...]
        y = y_ref[...]
        z = z_ref[...]
        onehot = (iota_f == farthest).astype(jnp.float32)
        cx = jnp.sum(x * onehot, axis=1, keepdims=True)
        cy = jnp.sum(y * onehot, axis=1, keepdims=True)
        cz = jnp.sum(z * onehot, axis=1, keepdims=True)
        slot = (iota_s == i).astype(jnp.float32)
        cxs = cxs + slot * cx
        cys = cys + slot * cy
        czs = czs + slot * cz
        dx = x - cx
        dy = y - cy
        dz = z - cz
        dist = (dx * dx + dy * dy) + dz * dz
        distance = jnp.minimum(distance, dist)
        maxv = jnp.max(distance, axis=1, keepdims=True)
        farthest = jnp.min(
            jnp.where(distance == maxv, iota_f, float(N)),
            axis=1, keepdims=True)
        return distance, farthest, cxs, cys, czs

    zs = jnp.zeros((B, S), jnp.float32)
    init = (jnp.full((B, N), 1e10, jnp.float32),
            jnp.zeros((B, 1), jnp.float32),
            zs, zs, zs)
    _, _, cxs, cys, czs = jax.lax.fori_loop(0, S, body, init)
    ox_ref[...] = cxs
    oy_ref[...] = cys
    oz_ref[...] = czs


def _fps_pallas(xyz, npoint):
    B, _, N = xyz.shape
    ox, oy, oz = pl.pallas_call(
        _fps_kernel,
        out_shape=(
            jax.ShapeDtypeStruct((B, npoint), jnp.float32),
            jax.ShapeDtypeStruct((B, npoint), jnp.float32),
            jax.ShapeDtypeStruct((B, npoint), jnp.float32),
        ),
    )(xyz[:, 0], xyz[:, 1], xyz[:, 2])
    return ox, oy, oz


_GCHUNK = 128


def _gather_sc(table, fidx0, fidx1, fidx2):
    D = table.shape[1]
    Rs = (fidx0.shape[0], fidx1.shape[0], fidx2.shape[0])
    mesh = plsc.VectorSubcoreMesh(core_axis_name="c", subcore_axis_name="s")
    cp = pltpu.CompilerParams()
    fields = pltpu.CompilerParams.__dataclass_fields__
    if "needs_layout_passes" in fields:
        cp = dataclasses.replace(cp, needs_layout_passes=False)
    if "use_tc_tiling_on_sc" in fields:
        cp = dataclasses.replace(cp, use_tc_tiling_on_sc=False)

    @functools.partial(
        pl.kernel,
        compiler_params=cp,
        out_type=tuple(jax.ShapeDtypeStruct((r, D), jnp.float32) for r in Rs),
        mesh=mesh,
        scratch_types=[
            pltpu.VMEM((_GCHUNK,), jnp.int32),
            pltpu.VMEM((_GCHUNK, D), jnp.float32),
            pltpu.SemaphoreType.DMA,
        ],
    )
    def gkern(tab_hbm, i0_hbm, i1_hbm, i2_hbm, o0_hbm, o1_hbm, o2_hbm,
              idxbuf, gbuf, sem):
        wid = lax.axis_index("s") * 2 + lax.axis_index("c")
        for i_hbm, o_hbm, r_total in ((i0_hbm, o0_hbm, Rs[0]),
                                      (i1_hbm, o1_hbm, Rs[1]),
                                      (i2_hbm, o2_hbm, Rs[2])):
            rows_per = r_total // 32
            base = wid * rows_per

            @pl.loop(0, rows_per // _GCHUNK)
            def _chunk(c, base=base, i_hbm=i_hbm, o_hbm=o_hbm):
                row0 = base + c * _GCHUNK
                pltpu.sync_copy(i_hbm.at[pl.ds(row0, _GCHUNK)], idxbuf)
                pltpu.async_copy(tab_hbm.at[idxbuf], gbuf, sem).wait()
                pltpu.sync_copy(gbuf, o_hbm.at[pl.ds(row0, _GCHUNK), :])

    return gkern(table, fidx0, fidx1, fidx2)


def kernel(xyz, points, Ws, bs):
    xyz_t = jnp.transpose(xyz, (0, 2, 1))
    pts_t = jnp.transpose(points, (0, 2, 1))
    B, N, C = xyz_t.shape
    S = _NPOINT
    cxs, cys, czs = _fps_pallas(xyz, S)
    new_xyz_cs = jnp.stack([cxs, cys, czs], axis=1)
    new_xyz = jnp.transpose(new_xyz_cs, (0, 2, 1))
    w0, w1, w2 = _ballmask_pallas(
        xyz[:, 0], xyz[:, 1], xyz[:, 2], cxs, cys, czs)
    g0, g1, g2 = _ballquery_sc(
        w0.reshape(B * S, 128), w1.reshape(B * S, 128),
        w2.reshape(B * S, 128))
    group_idx_all = [g0.reshape(B, S, 16), g1.reshape(B, S, 32),
                     g2.reshape(B, S, 64)]
    D_PAD = 80
    DF = pts_t.shape[-1]
    table = jnp.concatenate(
        [pts_t, xyz_t, jnp.zeros((B, N, D_PAD - DF - 3), jnp.float32)],
        axis=-1).reshape(B * N, D_PAD)
    boff = (jnp.arange(B, dtype=jnp.int32) * N)[:, None, None]
    fidx = [(g + boff).reshape(-1) for g in group_idx_all]
    G0, G1, G2 = _gather_sc(table, fidx[0], fidx[1], fidx[2])
    new_points_list = []
    for i, G in enumerate((G0, G1, G2)):
        K = _NSAMPLE_LIST[i]
        G = G.reshape(B, S, K, D_PAD)
        grouped_xyz = G[..., DF:DF + 3] - new_xyz[:, :, None, :]
        h = jnp.concatenate([G[..., :DF], grouped_xyz], axis=-1)
        for W, b in zip(Ws[i], bs[i]):
            h = jax.nn.relu(jnp.einsum('bskc,oc->bsko', h, W) + b)
        new_points = jnp.max(h, axis=2)
        new_points_list.append(jnp.transpose(new_points, (0, 2, 1)))
    new_points_concat = jnp.concatenate(new_points_list, axis=1)
    return new_xyz_cs, new_points_concat
